# baseline (speedup 1.0000x reference)
import functools

import jax
import jax.numpy as jnp
from jax import lax
from jax.experimental import pallas as pl
from jax.experimental.pallas import tpu as pltpu
from jax.experimental.pallas import tpu_sc as plsc

CHANNEL = 128
SIZE0, SIZE1 = 4, 4
EMB_DIM = CHANNEL * SIZE0 * SIZE1
BATCH = 16384
VOCAB = 10
NC, NS = 2, 16
NW = NC * NS
B_PER_W = BATCH // NW
CHUNK = 16
NCHUNK = B_PER_W // CHUNK
LANES = 16
VPR = EMB_DIM // LANES


_mesh = plsc.VectorSubcoreMesh(core_axis_name="c", subcore_axis_name="s")


@functools.partial(
    pl.kernel,
    out_type=jax.ShapeDtypeStruct((BATCH * EMB_DIM,), jnp.float32),
    mesh=_mesh,
    scratch_types=[
        pltpu.VMEM((B_PER_W,), jnp.int32),
        pltpu.VMEM((VOCAB * EMB_DIM,), jnp.float32),
        pltpu.VMEM((CHUNK * EMB_DIM,), jnp.float32),
        pltpu.VMEM((CHUNK * EMB_DIM,), jnp.float32),
        pltpu.SemaphoreType.DMA,
        pltpu.SemaphoreType.DMA,
    ],
)
def _lookup(digits_hbm, table_hbm, out_hbm, idx_v, table_v, buf0, buf1,
            sw0, sw1):
    wid = lax.axis_index("s") * NC + lax.axis_index("c")
    base = wid * B_PER_W

    pltpu.sync_copy(digits_hbm.at[pl.ds(base, B_PER_W)], idx_v)
    for v in range(VOCAB):
        pltpu.sync_copy(table_hbm.at[v], table_v.at[pl.ds(v * EMB_DIM, EMB_DIM)])

    bufs = (buf0, buf1)
    wsems = (sw0, sw1)

    def out_ref(g):
        return out_hbm.at[pl.ds((base + g * CHUNK) * EMB_DIM,
                                CHUNK * EMB_DIM)]

    def assemble(g, buf):
        rows_vec = idx_v[pl.ds(g * CHUNK, CHUNK)]
        for j in range(CHUNK):
            src_base = rows_vec[j] * EMB_DIM
            dst_base = j * EMB_DIM

            def cp(k, c, src_base=src_base, dst_base=dst_base):
                for u in range(16):
                    off = k * 256 + u * LANES
                    buf[pl.ds(dst_base + off, LANES)] = (
                        table_v[pl.ds(src_base + off, LANES)]
                    )
                return c

            lax.fori_loop(0, VPR // 16, cp, 0)

    def body(i, carry):
        for p in range(2):
            g = i * 2 + p

            @pl.when(g >= 2)
            def _wait_prev():
                pltpu.make_async_copy(
                    bufs[p], out_ref(0), wsems[p]
                ).wait()

            assemble(g, bufs[p])
            pltpu.async_copy(bufs[p], out_ref(g), wsems[p])
        return carry

    lax.fori_loop(0, NCHUNK // 2, body, 0)

    for p in range(2):
        pltpu.make_async_copy(bufs[p], out_ref(0), wsems[p]).wait()


def kernel(digits, table):
    out = _lookup(digits, table)
    return out.reshape(-1, CHANNEL, SIZE0, SIZE1)


# speedup vs baseline: 17.2114x; 17.2114x over previous
import functools

import jax
import jax.numpy as jnp
from jax import lax
from jax.experimental import pallas as pl
from jax.experimental.pallas import tpu as pltpu
from jax.experimental.pallas import tpu_sc as plsc

CHANNEL = 128
SIZE0, SIZE1 = 4, 4
EMB_DIM = CHANNEL * SIZE0 * SIZE1
BATCH = 16384
VOCAB = 10
NC, NS = 2, 16
NW = NC * NS
B_PER_W = BATCH // NW
NSEM = 4
GROUP = 16


_mesh = plsc.VectorSubcoreMesh(core_axis_name="c", subcore_axis_name="s")


@functools.partial(
    pl.kernel,
    out_type=jax.ShapeDtypeStruct((BATCH, EMB_DIM), jnp.float32),
    mesh=_mesh,
    scratch_types=[
        pltpu.VMEM((B_PER_W,), jnp.int32),
        pltpu.VMEM((VOCAB, EMB_DIM), jnp.float32),
        pltpu.SemaphoreType.DMA,
        pltpu.SemaphoreType.DMA,
        pltpu.SemaphoreType.DMA,
        pltpu.SemaphoreType.DMA,
    ],
)
def _lookup(digits_hbm, table_hbm, out_hbm, idx_v, table_v, s0, s1, s2, s3):
    wid = lax.axis_index("s") * NC + lax.axis_index("c")
    base = wid * B_PER_W
    sems = (s0, s1, s2, s3)

    pltpu.sync_copy(digits_hbm.at[pl.ds(base, B_PER_W)], idx_v)
    pltpu.sync_copy(table_hbm, table_v)

    def fire(g, carry):
        goff = g * GROUP
        vec = idx_v[pl.ds(goff, GROUP)]
        for k in range(GROUP):
            row = vec[k]
            pltpu.async_copy(
                table_v.at[pl.ds(row, 1)],
                out_hbm.at[pl.ds(base + goff + k, 1)],
                sems[k % NSEM],
            )
        return carry

    lax.fori_loop(0, B_PER_W // GROUP, fire, 0)

    def drain(j, carry):
        for p in range(NSEM):
            pltpu.make_async_copy(
                table_hbm.at[pl.ds(0, 8)], table_v.at[pl.ds(0, 8)], sems[p]
            ).wait()
        return carry

    lax.fori_loop(0, B_PER_W // NSEM // 8, drain, 0)


def kernel(digits, table):
    out = _lookup(digits, table)
    return out.reshape(-1, CHANNEL, SIZE0, SIZE1)
